# baseline (speedup 1.0000x reference)
import functools

import jax
import jax.numpy as jnp
from jax import lax
from jax.experimental import pallas as pl
from jax.experimental.pallas import tpu as pltpu
from jax.experimental.pallas import tpu_sc as plsc

_NCODES = 16384
_BATCH = 4096
_D = 512
_L = 16
_NC = 2
_NS = 16
_NW = _NC * _NS
_BPW = _BATCH // _NW
_NCHUNKS = _BATCH // _L
_PIPE = 4
_ROWS = _BPW // _PIPE


def _body(idx_hbm, data_hbm, out_hbm, idx_v, table_v, src_v,
          buf0, buf1, buf2, buf3, gs0, gs1, gs2, gs3, ws0, ws1, ws2, ws3):
    bufs = (buf0, buf1, buf2, buf3)
    gsems = (gs0, gs1, gs2, gs3)
    wsems = (ws0, ws1, ws2, ws3)
    wid = lax.axis_index("s") * _NC + lax.axis_index("c")

    with jax.named_scope("stage_idx"):
        pltpu.sync_copy(idx_hbm, idx_v)

    lane = lax.iota(jnp.int32, _L)
    nxt_lane = (lane + 1) & (_L - 1)
    last_lane = lane == (_L - 1)

    def chunk_step(c, carry):
        chunk = idx_v[pl.ds(c * _L, _L)]
        comp = chunk * _L + lane
        sk, _ = plsc.sort_key_val(comp, comp)
        nxt = jnp.take(sk, nxt_lane, mode="wrap")
        code = sk >> 4
        is_last = jnp.logical_or(code != (nxt >> 4), last_lane)
        pos = (sk & (_L - 1)) + c * _L
        plsc.store_scatter(table_v, [code], pos, mask=is_last)
        return carry

    with jax.named_scope("phaseA_table"):
        lax.fori_loop(0, _NCHUNKS, chunk_step, 0, unroll=8)

    base = wid * _BPW
    with jax.named_scope("phaseB_srcs"):
        for b in range(_BPW // _L):
            my = idx_v[pl.ds(base + b * _L, _L)]
            src_v[pl.ds(b * _L, _L)] = plsc.load_gather(table_v, [my])

    with jax.named_scope("phaseB_rows"):
        gets = [
            pltpu.async_copy(
                data_hbm.at[src_v.at[pl.ds(k * _ROWS, _ROWS)]],
                bufs[k], gsems[k])
            for k in range(_PIPE)
        ]
        puts = []
        for k in range(_PIPE):
            gets[k].wait()
            puts.append(pltpu.async_copy(
                bufs[k], out_hbm.at[pl.ds(base + k * _ROWS, _ROWS)],
                wsems[k]))
        for p in puts:
            p.wait()


_cache_lookup = functools.partial(
    pl.kernel,
    out_type=jax.ShapeDtypeStruct((_BATCH, _D), jnp.float32),
    mesh=plsc.VectorSubcoreMesh(
        core_axis_name="c", subcore_axis_name="s",
        num_cores=_NC, num_subcores=_NS),
    scratch_types=[
        pltpu.VMEM((_BATCH,), jnp.int32),
        pltpu.VMEM((_NCODES,), jnp.int32),
        pltpu.VMEM((_BPW,), jnp.int32),
        pltpu.VMEM((_ROWS, _D), jnp.float32),
        pltpu.VMEM((_ROWS, _D), jnp.float32),
        pltpu.VMEM((_ROWS, _D), jnp.float32),
        pltpu.VMEM((_ROWS, _D), jnp.float32),
        pltpu.SemaphoreType.DMA,
        pltpu.SemaphoreType.DMA,
        pltpu.SemaphoreType.DMA,
        pltpu.SemaphoreType.DMA,
        pltpu.SemaphoreType.DMA,
        pltpu.SemaphoreType.DMA,
        pltpu.SemaphoreType.DMA,
        pltpu.SemaphoreType.DMA,
    ],
    compiler_params=pltpu.CompilerParams(needs_layout_passes=False),
)(_body)


@jax.jit
def kernel(seq_idxs, set_data, cache):
    del cache
    return _cache_lookup(seq_idxs.astype(jnp.int32), set_data)
